# Initial kernel scaffold; baseline (speedup 1.0000x reference)
import jax
import jax.numpy as jnp
from jax.experimental import pallas as pl


def kernel(x, values, biases, rows, cols):
    raise NotImplementedError("write your pallas kernel here")


# speedup vs baseline: 4.9090x; 4.9090x over previous
import functools

import jax
import jax.numpy as jnp
from jax import lax
from jax.experimental import pallas as pl
from jax.experimental.pallas import tpu as pltpu
from jax.experimental.pallas import tpu_sc as plsc

NC = 2
NS = 16
NW = NC * NS
L = 16
C = 128


def _vreg_gather(vec, idx):
    dnums = lax.GatherDimensionNumbers(
        offset_dims=(), collapsed_slice_dims=(0,), start_index_map=(0,))
    return lax.gather(vec, idx[:, None], dnums, slice_sizes=(1,),
                      mode=lax.GatherScatterMode.PROMISE_IN_BOUNDS)


def _transpose_body(x_ref, o_ref):
    o_ref[...] = x_ref[...].T


def _combine_body(p_ref, b_ref, o_ref):
    s = p_ref[0] + p_ref[1]
    o_ref[...] = s.T + b_ref[...][None, :]


def _sc_spmm_body(nchunks, xt_hbm, val_hbm, row_hbm, col_hbm, out_hbm,
                  acc, colv, rowv, valv, gbuf, zbuf, sem):
    n = acc.shape[0]
    rps = n // NS
    cid = lax.axis_index("c")
    sid = lax.axis_index("s")
    wid = cid * NS + sid
    epw = nchunks * C

    def zero_row(i, _):
        for j in range(4):
            zbuf[i, pl.ds(L * j, L)] = jnp.zeros((L,), jnp.float32)
        return _
    lax.fori_loop(0, C, zero_row, None)
    for k in range(rps // C):
        pltpu.sync_copy(zbuf, acc.at[pl.ds(sid * rps + k * C, C)])
    plsc.subcore_barrier()

    def chunk(t, _):
        base = wid * epw + t * C
        pltpu.sync_copy(col_hbm.at[pl.ds(base, C)], colv)
        pltpu.sync_copy(val_hbm.at[pl.ds(base, C)], valv)
        pltpu.sync_copy(row_hbm.at[pl.ds(base, C)], rowv)
        pltpu.async_copy(xt_hbm.at[colv], gbuf, sem).wait()

        def group(g, _):
            vals16 = valv[pl.ds(g * L, L)]
            for e in range(L):
                bv = _vreg_gather(vals16, jnp.full((L,), e, jnp.int32))
                row = g * L + e
                for j in range(4):
                    gbuf[row, pl.ds(L * j, L)] = (
                        gbuf[row, pl.ds(L * j, L)] * bv)
            return _
        lax.fori_loop(0, C // L, group, None)

        pltpu.sync_copy(gbuf, acc.at[rowv], add=True)
        return _
    lax.fori_loop(0, nchunks, chunk, None)

    plsc.subcore_barrier()
    pltpu.sync_copy(acc.at[pl.ds(sid * rps, rps)],
                    out_hbm.at[cid, pl.ds(sid * rps, rps)])


def kernel(x, values, biases, rows, cols):
    b, n = x.shape
    nnz = values.shape[0]

    nchunks = -(-nnz // (NW * C))
    nnz_pad = nchunks * C * NW
    pad = nnz_pad - nnz
    valp = jnp.concatenate([values, jnp.zeros((pad,), values.dtype)])
    rowp = jnp.concatenate([rows, jnp.zeros((pad,), rows.dtype)])
    colp = jnp.concatenate([cols, jnp.zeros((pad,), cols.dtype)])

    blk = 512
    xt = pl.pallas_call(
        _transpose_body,
        grid=(n // blk,),
        in_specs=[pl.BlockSpec((b, blk), lambda i: (0, i))],
        out_specs=pl.BlockSpec((blk, b), lambda i: (i, 0)),
        out_shape=jax.ShapeDtypeStruct((n, b), jnp.float32),
    )(x)

    sc_spmm = functools.partial(
        pl.kernel,
        functools.partial(_sc_spmm_body, nchunks),
        out_type=jax.ShapeDtypeStruct((NC, n, b), jnp.float32),
        mesh=plsc.VectorSubcoreMesh(core_axis_name="c",
                                    subcore_axis_name="s"),
        compiler_params=pltpu.CompilerParams(use_tc_tiling_on_sc=False),
        scratch_types=[
            pltpu.VMEM_SHARED((n, b), jnp.float32),
            pltpu.VMEM((C,), jnp.int32),
            pltpu.VMEM((C,), jnp.int32),
            pltpu.VMEM((C,), jnp.float32),
            pltpu.VMEM((C, b), jnp.float32),
            pltpu.VMEM((C, b), jnp.float32),
            pltpu.SemaphoreType.DMA,
        ],
    )()
    partials = sc_spmm(xt, valp, rowp, colp)

    out = pl.pallas_call(
        _combine_body,
        grid=(n // blk,),
        in_specs=[
            pl.BlockSpec((NC, blk, b), lambda i: (0, i, 0)),
            pl.BlockSpec((blk,), lambda i: (i,)),
        ],
        out_specs=pl.BlockSpec((b, blk), lambda i: (0, i)),
        out_shape=jax.ShapeDtypeStruct((b, n), jnp.float32),
    )(partials, biases)
    return out


# speedup vs baseline: 5.3539x; 1.0906x over previous
import functools

import jax
import jax.numpy as jnp
from jax import lax
from jax.experimental import pallas as pl
from jax.experimental.pallas import tpu as pltpu
from jax.experimental.pallas import tpu_sc as plsc

NC = 2
NS = 16
NW = NC * NS
L = 16
C = 128
G = 4


def _vreg_gather(vec, idx):
    dnums = lax.GatherDimensionNumbers(
        offset_dims=(), collapsed_slice_dims=(0,), start_index_map=(0,))
    return lax.gather(vec, idx[:, None], dnums, slice_sizes=(1,),
                      mode=lax.GatherScatterMode.PROMISE_IN_BOUNDS)


def _transpose_body(x_ref, o_ref):
    o_ref[...] = x_ref[...].T


def _combine_body(p_ref, b_ref, o_ref):
    s = p_ref[0] + p_ref[1]
    o_ref[...] = s.T + b_ref[...][None, :]


def _sc_spmm_body(nchunks, xt_hbm, val_hbm, row_hbm, col_hbm, out_hbm,
                  acc, colv, rowv, valv, gbufs, sload, sg, ss):
    n = acc.shape[0]
    rps = n // NS
    cid = lax.axis_index("c")
    sid = lax.axis_index("s")
    wid = cid * NS + sid

    dc = pltpu.async_copy(col_hbm.at[wid], colv, sload)
    dr = pltpu.async_copy(row_hbm.at[wid], rowv, sload)
    dv = pltpu.async_copy(val_hbm.at[wid], valv, sload)

    def zero_row(i, _):
        for j in range(4):
            gbufs[0, i, pl.ds(L * j, L)] = jnp.zeros((L,), jnp.float32)
        return _
    lax.fori_loop(0, C, zero_row, None)
    for k in range(rps // C):
        pltpu.sync_copy(gbufs.at[0], acc.at[pl.ds(sid * rps + k * C, C)])
    dc.wait(); dr.wait(); dv.wait()
    plsc.subcore_barrier()

    def gather(t, b):
        return pltpu.async_copy(xt_hbm.at[colv.at[t]], gbufs.at[b], sg[b])

    def scatter(t, b):
        return pltpu.async_copy(gbufs.at[b], acc.at[rowv.at[t]], ss[b],
                                add=True)

    for b in range(G):
        gather(b, b)

    nq = nchunks // G

    def round_(q, _):
        descs = []
        for b in range(G):
            t = q * G + b
            pltpu.make_async_copy(xt_hbm.at[colv.at[t]], gbufs.at[b],
                                  sg[b]).wait()

            def group(g, _):
                vals16 = valv[t, pl.ds(g * L, L)]
                for e in range(L):
                    bv = _vreg_gather(vals16, jnp.full((L,), e, jnp.int32))
                    row = g * L + e
                    for j in range(4):
                        gbufs[b, row, pl.ds(L * j, L)] = (
                            gbufs[b, row, pl.ds(L * j, L)] * bv)
                return _
            lax.fori_loop(0, C // L, group, None)
            descs.append(scatter(t, b))
        for b in range(G):
            descs[b].wait()

            @pl.when(q < nq - 1)
            def _():
                gather(q * G + b + G, b)
        return _
    lax.fori_loop(0, nq, round_, None)

    plsc.subcore_barrier()
    pltpu.sync_copy(acc.at[pl.ds(sid * rps, rps)],
                    out_hbm.at[cid, pl.ds(sid * rps, rps)])


def kernel(x, values, biases, rows, cols):
    b, n = x.shape
    nnz = values.shape[0]

    nchunks = -(-nnz // (NW * C * G)) * G
    nnz_pad = nchunks * C * NW
    pad = nnz_pad - nnz
    valp = jnp.concatenate([values, jnp.zeros((pad,), values.dtype)])
    rowp = jnp.concatenate([rows, jnp.zeros((pad,), rows.dtype)])
    colp = jnp.concatenate([cols, jnp.zeros((pad,), cols.dtype)])
    valp = valp.reshape(NW, nchunks, C)
    rowp = rowp.reshape(NW, nchunks, C)
    colp = colp.reshape(NW, nchunks, C)

    blk = 512
    xt = pl.pallas_call(
        _transpose_body,
        grid=(n // blk,),
        in_specs=[pl.BlockSpec((b, blk), lambda i: (0, i))],
        out_specs=pl.BlockSpec((blk, b), lambda i: (i, 0)),
        out_shape=jax.ShapeDtypeStruct((n, b), jnp.float32),
    )(x)

    sc_spmm = pl.kernel(
        functools.partial(_sc_spmm_body, nchunks),
        out_type=jax.ShapeDtypeStruct((NC, n, b), jnp.float32),
        mesh=plsc.VectorSubcoreMesh(core_axis_name="c",
                                    subcore_axis_name="s"),
        compiler_params=pltpu.CompilerParams(use_tc_tiling_on_sc=False),
        scratch_types=[
            pltpu.VMEM_SHARED((n, b), jnp.float32),
            pltpu.VMEM((nchunks, C), jnp.int32),
            pltpu.VMEM((nchunks, C), jnp.int32),
            pltpu.VMEM((nchunks, C), jnp.float32),
            pltpu.VMEM((G, C, b), jnp.float32),
            pltpu.SemaphoreType.DMA,
            [pltpu.SemaphoreType.DMA] * G,
            [pltpu.SemaphoreType.DMA] * G,
        ],
    )
    partials = sc_spmm(xt, valp, rowp, colp)

    out = pl.pallas_call(
        _combine_body,
        grid=(n // blk,),
        in_specs=[
            pl.BlockSpec((NC, blk, b), lambda i: (0, i, 0)),
            pl.BlockSpec((blk,), lambda i: (i,)),
        ],
        out_specs=pl.BlockSpec((b, blk), lambda i: (0, i)),
        out_shape=jax.ShapeDtypeStruct((b, n), jnp.float32),
    )(partials, biases)
    return out


# speedup vs baseline: 8.3068x; 1.5515x over previous
import functools

import jax
import jax.numpy as jnp
import numpy as np
from jax import lax
from jax.experimental import pallas as pl
from jax.experimental.pallas import tpu as pltpu
from jax.experimental.pallas import tpu_sc as plsc

NC = 2
NS = 16
NW = NC * NS
L = 16
C = 128
G = 4
S = 2


def _vreg_gather(vec, idx):
    dnums = lax.GatherDimensionNumbers(
        offset_dims=(), collapsed_slice_dims=(0,), start_index_map=(0,))
    return lax.gather(vec, idx[:, None], dnums, slice_sizes=(1,),
                      mode=lax.GatherScatterMode.PROMISE_IN_BOUNDS)


def _transpose_body(x_ref, o_ref):
    o_ref[...] = x_ref[...].T.astype(jnp.bfloat16)


def _combine_body(p_ref, b_ref, o_ref):
    s = p_ref[0] + p_ref[1]
    o_ref[...] = s.T + b_ref[...][None, :]


def _scale_unpacked(gbufs, sbufs, valv, t, b, s):
    def group(g, _):
        vals16 = valv[t, pl.ds(g * L, L)]
        for e in range(L):
            bv = _vreg_gather(vals16, jnp.full((L,), e, jnp.int32))
            row = g * L + e
            for j in range(2):
                xi = gbufs[b, row, pl.ds(j * L, L)]
                lo = plsc.bitcast(lax.shift_left(xi, 16), jnp.float32)
                hi = plsc.bitcast(
                    lax.bitwise_and(xi, jnp.int32(-65536)), jnp.float32)
                sbufs[s, row, pl.ds(j * 2 * L, L)] = lo * bv
                sbufs[s, row, pl.ds(j * 2 * L + L, L)] = hi * bv
        return _
    lax.fori_loop(0, C // L, group, None)


def _sc_spmm_body(nchunks, xt_hbm, val_hbm, row_hbm, col_hbm, out_hbm,
                  acc, colv, rowv, valv, gbufs, sbufs, sload, sg, ss):
    n = acc.shape[0]
    rps = n // NS
    cid = lax.axis_index("c")
    sid = lax.axis_index("s")
    wid = cid * NS + sid

    dc = pltpu.async_copy(col_hbm.at[wid], colv, sload)
    dr = pltpu.async_copy(row_hbm.at[wid], rowv, sload)
    dv = pltpu.async_copy(val_hbm.at[wid], valv, sload)

    def zero_row(i, _):
        for j in range(4):
            sbufs[0, i, pl.ds(L * j, L)] = jnp.zeros((L,), jnp.float32)
        return _
    lax.fori_loop(0, C, zero_row, None)
    for k in range(rps // C):
        pltpu.sync_copy(sbufs.at[0], acc.at[pl.ds(sid * rps + k * C, C)])
    dc.wait(); dr.wait(); dv.wait()
    plsc.subcore_barrier()

    def gather(t, b):
        return pltpu.async_copy(xt_hbm.at[colv.at[t]], gbufs.at[b], sg[b])

    def scatter(t, s):
        return pltpu.async_copy(sbufs.at[s], acc.at[rowv.at[t]], ss[s],
                                add=True)

    def scatter_wait(s):
        pltpu.make_async_copy(sbufs.at[s], acc.at[rowv.at[0]],
                              ss[s]).wait()

    for b in range(G):
        gather(b, b)

    nq = nchunks // G

    def round_(q, _):
        for b in range(G):
            t = q * G + b
            s = b % S
            pltpu.make_async_copy(xt_hbm.at[colv.at[t]], gbufs.at[b],
                                  sg[b]).wait()
            if b < S:
                @pl.when(q > 0)
                def _():
                    scatter_wait(s)
            else:
                scatter_wait(s)
            _scale_unpacked(gbufs, sbufs, valv, t, b, s)
            scatter(t, s)

            @pl.when(q < nq - 1)
            def _():
                gather(q * G + b + G, b)
        return _
    lax.fori_loop(0, nq, round_, None)

    for s in range(S):
        scatter_wait(s)
    plsc.subcore_barrier()
    pltpu.sync_copy(acc.at[pl.ds(sid * rps, rps)],
                    out_hbm.at[cid, pl.ds(sid * rps, rps)])


def kernel(x, values, biases, rows, cols):
    b, n = x.shape
    nnz = values.shape[0]

    m = np.arange(b)
    perm = (m // 32) * 32 + (m % 2) * 16 + (m % 32) // 2
    x2 = x[jnp.asarray(perm), :]

    nchunks = -(-nnz // (NW * C * G)) * G
    nnz_pad = nchunks * C * NW
    pad = nnz_pad - nnz
    valp = jnp.concatenate([values, jnp.zeros((pad,), values.dtype)])
    rowp = jnp.concatenate([rows, jnp.zeros((pad,), rows.dtype)])
    colp = jnp.concatenate([cols, jnp.zeros((pad,), cols.dtype)])
    valp = valp.reshape(NW, nchunks, C)
    rowp = rowp.reshape(NW, nchunks, C)
    colp = colp.reshape(NW, nchunks, C)

    blk = 512
    xt = pl.pallas_call(
        _transpose_body,
        grid=(n // blk,),
        in_specs=[pl.BlockSpec((b, blk), lambda i: (0, i))],
        out_specs=pl.BlockSpec((blk, b), lambda i: (i, 0)),
        out_shape=jax.ShapeDtypeStruct((n, b), jnp.bfloat16),
    )(x2)
    xt = lax.bitcast_convert_type(xt.reshape(n, b // 2, 2), jnp.int32)

    sc_spmm = pl.kernel(
        functools.partial(_sc_spmm_body, nchunks),
        out_type=jax.ShapeDtypeStruct((NC, n, b), jnp.float32),
        mesh=plsc.VectorSubcoreMesh(core_axis_name="c",
                                    subcore_axis_name="s"),
        compiler_params=pltpu.CompilerParams(use_tc_tiling_on_sc=False,
                                             needs_layout_passes=False),
        scratch_types=[
            pltpu.VMEM_SHARED((n, b), jnp.float32),
            pltpu.VMEM((nchunks, C), jnp.int32),
            pltpu.VMEM((nchunks, C), jnp.int32),
            pltpu.VMEM((nchunks, C), jnp.float32),
            pltpu.VMEM((G, C, b // 2), jnp.int32),
            pltpu.VMEM((S, C, b), jnp.float32),
            pltpu.SemaphoreType.DMA,
            [pltpu.SemaphoreType.DMA] * G,
            [pltpu.SemaphoreType.DMA] * S,
        ],
    )
    partials = sc_spmm(xt, valp, rowp, colp)

    out = pl.pallas_call(
        _combine_body,
        grid=(n // blk,),
        in_specs=[
            pl.BlockSpec((NC, blk, b), lambda i: (0, i, 0)),
            pl.BlockSpec((blk,), lambda i: (i,)),
        ],
        out_specs=pl.BlockSpec((b, blk), lambda i: (0, i)),
        out_shape=jax.ShapeDtypeStruct((b, n), jnp.float32),
    )(partials, biases)
    return out
